# Initial kernel scaffold; baseline (speedup 1.0000x reference)
import jax
import jax.numpy as jnp
from jax.experimental import pallas as pl


def kernel(inputs, adj, emb, W1, a_src, a_dst, W2, ao_src, ao_dst):
    raise NotImplementedError("write your pallas kernel here")


# speedup vs baseline: 2.5159x; 2.5159x over previous
import jax
import jax.numpy as jnp
from jax import lax
from jax.experimental import pallas as pl
from jax.experimental.pallas import tpu as pltpu
from jax.experimental.pallas import tpu_sc as plsc

N = 10000
E = 160000
V = 100000
L = 50
D = 128
H = 8
HID = 64
C = 42
NEG = 0.2

NC = 2
NS = 16
NW = NC * NS
NP = 10240
NODES_PER_TILE = NP // NW
ECH = 128
ECH_PER_TILE = (E + NW * ECH - 1) // (NW * ECH)
EPAD = ECH_PER_TILE * NW * ECH
CPAD = 48
GW = 128
f32 = jnp.float32
i32 = jnp.int32

TBLK = 2048

_SC_PARAMS = pltpu.CompilerParams(needs_layout_passes=False)


def _sc_mesh():
    return plsc.VectorSubcoreMesh(core_axis_name="c", subcore_axis_name="s")


def _emb_body(emb_hbm, inp_hbm, enc_hbm, inp_v, rows_v, enc_v, sem):
    c = lax.axis_index("c")
    s = lax.axis_index("s")
    wid = s * NC + c
    base = wid * NODES_PER_TILE
    pltpu.sync_copy(inp_hbm.at[pl.ds(base, NODES_PER_TILE)], inp_v)

    def node_body(n, carry):
        pltpu.async_copy(emb_hbm.at[inp_v.at[n]], rows_v, sem).wait()

        def row_body(r, acc):
            return tuple(acc[q] + rows_v[r, pl.ds(q * 16, 16)]
                         for q in range(D // 16))

        acc0 = tuple(jnp.zeros((16,), f32) for _ in range(D // 16))
        acc = lax.fori_loop(0, L, row_body, acc0)
        for q in range(D // 16):
            enc_v[n, pl.ds(q * 16, 16)] = acc[q] * (1.0 / L)
        return carry

    lax.fori_loop(0, NODES_PER_TILE, node_body, 0)
    pltpu.sync_copy(enc_v, enc_hbm.at[pl.ds(base, NODES_PER_TILE)])


def _emb_pool(emb, inputs_pad):
    k = pl.kernel(
        _emb_body,
        out_type=jax.ShapeDtypeStruct((NP, D), f32),
        mesh=_sc_mesh(),
        compiler_params=_SC_PARAMS,
        scratch_types=[
            pltpu.VMEM((NODES_PER_TILE, L), i32),
            pltpu.VMEM((L, D), f32),
            pltpu.VMEM((NODES_PER_TILE, D), f32),
            pltpu.SemaphoreType.DMA,
        ],
    )
    return k(emb, inputs_pad)


def _make_sd_body(scol):
    def body(dst_hbm, ext_hbm, out_hbm, didx_v, drows_v, dsc_v, sem):
        c = lax.axis_index("c")
        s = lax.axis_index("s")
        wid = s * NC + c
        pltpu.sync_copy(dst_hbm.at[pl.ds(wid * ECH_PER_TILE, ECH_PER_TILE)],
                        didx_v)
        onehots = [(lax.iota(i32, 16) == k).astype(f32) for k in range(16)]

        def chunk_body(j, carry):
            pltpu.async_copy(ext_hbm.at[didx_v.at[j]], drows_v, sem).wait()
            for g in range(ECH // 16):
                evec = jnp.zeros((16,), f32)
                for k in range(16):
                    ed = jnp.sum(
                        drows_v[g * 16 + k, pl.ds(scol, 16)] * onehots[1],
                        axis=0)
                    evec = evec + lax.broadcast(ed, (16,)) * onehots[k]
                dsc_v[j, pl.ds(g * 16, 16)] = evec
            return carry

        lax.fori_loop(0, ECH_PER_TILE, chunk_body, 0)
        pltpu.sync_copy(dsc_v,
                        out_hbm.at[pl.ds(wid * ECH_PER_TILE, ECH_PER_TILE)])

    return body


def _sd_pass(dst2d, ext, scol):
    k = pl.kernel(
        _make_sd_body(scol),
        out_type=jax.ShapeDtypeStruct((NW * ECH_PER_TILE, ECH), f32),
        mesh=_sc_mesh(),
        compiler_params=_SC_PARAMS,
        scratch_types=[
            pltpu.VMEM((ECH_PER_TILE, ECH), i32),
            pltpu.VMEM((ECH, GW), f32),
            pltpu.VMEM((ECH_PER_TILE, ECH), f32),
            pltpu.SemaphoreType.DMA,
        ],
    )
    return k(dst2d, ext)


def _make_edge_body(uw, scol, ow):
    def body(adj_hbm, esc_hbm, ext_hbm, out_hbm,
             aidx_v, rows_v, drows_v, orows_v, esc_v, acc_sh, sem, semb):
        c = lax.axis_index("c")
        s = lax.axis_index("s")
        wid = s * NC + c
        pltpu.sync_copy(adj_hbm.at[pl.ds(wid * 2 * ECH_PER_TILE,
                                         2 * ECH_PER_TILE)], aidx_v)
        pltpu.sync_copy(esc_hbm.at[pl.ds(wid * ECH_PER_TILE, ECH_PER_TILE)],
                        esc_v)

        def zrow(j, carry):
            for q in range(ow // 16):
                orows_v[j, pl.ds(q * 16, 16)] = jnp.zeros((16,), f32)
            return carry

        lax.fori_loop(0, ECH, zrow, 0)
        rows_per_sub = NP // NS
        for i in range(rows_per_sub // ECH):
            pltpu.sync_copy(
                orows_v, acc_sh.at[pl.ds(s * rows_per_sub + i * ECH, ECH)])
        plsc.subcore_barrier()

        onehots = [(lax.iota(i32, 16) == k).astype(f32) for k in range(16)]

        def chunk_body(j, carry):
            pltpu.async_copy(ext_hbm.at[aidx_v.at[j]], rows_v, sem).wait()
            for e in range(ECH):
                es = jnp.sum(rows_v[e, pl.ds(scol, 16)] * onehots[0], axis=0)
                ed = jnp.sum(
                    esc_v[j, pl.ds((e // 16) * 16, 16)] * onehots[e % 16],
                    axis=0)
                ev = lax.broadcast(es + ed, (16,))
                ev = jnp.where(ev >= 0, ev, NEG * ev)
                ae = jnp.exp(ev)
                for q in range(uw // 16):
                    orows_v[e, pl.ds(q * 16, 16)] = (
                        rows_v[e, pl.ds(q * 16, 16)] * ae)
                orows_v[e, pl.ds(uw, 16)] = ae * onehots[0]
            pltpu.sync_copy(orows_v,
                            acc_sh.at[aidx_v.at[ECH_PER_TILE + j]], add=True)
            return carry

        lax.fori_loop(0, ECH_PER_TILE, chunk_body, 0)
        plsc.subcore_barrier()
        for i in range(rows_per_sub // ECH):
            r0 = s * rows_per_sub + i * ECH
            pltpu.sync_copy(acc_sh.at[pl.ds(r0, ECH)],
                            out_hbm.at[c, pl.ds(r0, ECH)])

    return body


def _edge_pass(adj2d, esc, ext, uw, scol):
    ow = uw + 16
    k = pl.kernel(
        _make_edge_body(uw, scol, ow),
        out_type=jax.ShapeDtypeStruct((NC, NP, ow), f32),
        mesh=_sc_mesh(),
        compiler_params=_SC_PARAMS,
        scratch_types=[
            pltpu.VMEM((2 * ECH_PER_TILE, ECH), i32),
            pltpu.VMEM((ECH, GW), f32),
            pltpu.VMEM((ECH, GW), f32),
            pltpu.VMEM((ECH, ow), f32),
            pltpu.VMEM((ECH_PER_TILE, ECH), f32),
            pltpu.VMEM_SHARED((NP, ow), f32),
            pltpu.SemaphoreType.DMA,
            pltpu.SemaphoreType.DMA,
        ],
    )
    return k(adj2d, esc, ext)


def _t1_body(enc_ref, w1_ref, a1_ref, *outs):
    wh = jnp.dot(enc_ref[...], w1_ref[...], preferred_element_type=f32)
    s16 = jnp.dot(wh, a1_ref[...], preferred_element_type=f32)
    z = jnp.zeros((wh.shape[0], GW - HID - 2), f32)
    for h in range(H):
        outs[h][...] = jnp.concatenate(
            [wh[:, h * HID:(h + 1) * HID], s16[:, 2 * h:2 * h + 2], z],
            axis=1)


def _t1(enc_pad, w1flat, a1mat):
    return pl.pallas_call(
        _t1_body,
        grid=(NP // TBLK,),
        in_specs=[
            pl.BlockSpec((TBLK, D), lambda i: (i, 0)),
            pl.BlockSpec((D, H * HID), lambda i: (0, 0)),
            pl.BlockSpec((H * HID, 16), lambda i: (0, 0)),
        ],
        out_specs=[pl.BlockSpec((TBLK, GW), lambda i: (i, 0))
                   for _ in range(H)],
        out_shape=[jax.ShapeDtypeStruct((NP, GW), f32) for _ in range(H)],
    )(enc_pad, w1flat, a1mat)


def _elu(x):
    return jnp.where(x >= 0, x, jnp.exp(jnp.minimum(x, 0.0)) - 1.0)


def _t2_body(w2_ref, a2_ref, *refs):
    acc_refs = refs[:H]
    ext2_out = refs[H]
    xs = []
    for h in range(H):
        a = acc_refs[h]
        agg = a[0, :, 0:HID] + a[1, :, 0:HID]
        den = a[0, :, HID:HID + 1] + a[1, :, HID:HID + 1] + 1e-10
        xs.append(_elu(agg / den))
    x1 = jnp.concatenate(xs, axis=1)
    wh2 = jnp.dot(x1, w2_ref[...], preferred_element_type=f32)
    s2 = jnp.dot(wh2, a2_ref[...], preferred_element_type=f32)
    z = jnp.zeros((wh2.shape[0], GW - CPAD - 2), f32)
    ext2_out[...] = jnp.concatenate(
        [wh2[:, 0:CPAD], s2[:, 0:2], z], axis=1)


def _t2(w2pad, a2mat, accs):
    ow = HID + 16
    return pl.pallas_call(
        _t2_body,
        grid=(NP // TBLK,),
        in_specs=[
            pl.BlockSpec((H * HID, GW), lambda i: (0, 0)),
            pl.BlockSpec((GW, 16), lambda i: (0, 0)),
        ] + [pl.BlockSpec((NC, TBLK, ow), lambda i: (0, i, 0))
             for _ in range(H)],
        out_specs=pl.BlockSpec((TBLK, GW), lambda i: (i, 0)),
        out_shape=jax.ShapeDtypeStruct((NP, GW), f32),
    )(w2pad, a2mat, *accs)


def _t3_body(acc_ref, out_ref):
    agg = acc_ref[0, :, 0:CPAD] + acc_ref[1, :, 0:CPAD]
    den = acc_ref[0, :, CPAD:CPAD + 1] + acc_ref[1, :, CPAD:CPAD + 1] + 1e-10
    h2 = _elu(agg / den)
    col = lax.broadcasted_iota(i32, h2.shape, 1)
    hm = jnp.where(col < C, h2, -1e30)
    m = jnp.max(hm, axis=1, keepdims=True)
    se = jnp.sum(jnp.exp(hm - m), axis=1, keepdims=True)
    out_ref[...] = h2 - m - jnp.log(se)


def _t3(acc2):
    ow = CPAD + 16
    return pl.pallas_call(
        _t3_body,
        grid=(NP // TBLK,),
        in_specs=[pl.BlockSpec((NC, TBLK, ow), lambda i: (0, i, 0))],
        out_specs=pl.BlockSpec((TBLK, CPAD), lambda i: (i, 0)),
        out_shape=jax.ShapeDtypeStruct((NP, CPAD), f32),
    )(acc2)


def kernel(inputs, adj, emb, W1, a_src, a_dst, W2, ao_src, ao_dst):
    src = adj[0]
    dst = adj[1]
    inputs_pad = jnp.pad(inputs, ((0, NP - N), (0, 0)))
    src3 = jnp.pad(src, (0, EPAD - E)).reshape(NW, ECH_PER_TILE, ECH)
    dst3 = jnp.pad(dst, (0, EPAD - E),
                   constant_values=NP - 1).reshape(NW, ECH_PER_TILE, ECH)
    adj2d = jnp.concatenate([src3, dst3], axis=1).reshape(
        NW * 2 * ECH_PER_TILE, ECH)

    enc_pad = _emb_pool(emb, inputs_pad)

    w1flat = jnp.transpose(W1, (1, 0, 2)).reshape(D, H * HID)
    a1mat = jnp.zeros((H * HID, 16), f32)
    for h in range(H):
        a1mat = a1mat.at[h * HID:(h + 1) * HID, 2 * h].set(a_src[h])
        a1mat = a1mat.at[h * HID:(h + 1) * HID, 2 * h + 1].set(a_dst[h])

    exts = _t1(enc_pad, w1flat, a1mat)

    dst2d = dst3.reshape(NW * ECH_PER_TILE, ECH)
    accs = []
    for h in range(H):
        esc = _sd_pass(dst2d, exts[h], HID)
        accs.append(_edge_pass(adj2d, esc, exts[h], HID, HID))

    w2pad = jnp.pad(W2, ((0, 0), (0, GW - C)))
    a2mat = jnp.zeros((GW, 16), f32)
    a2mat = a2mat.at[0:C, 0].set(ao_src)
    a2mat = a2mat.at[0:C, 1].set(ao_dst)

    ext2 = _t2(w2pad, a2mat, accs)
    esc2 = _sd_pass(dst2d, ext2, CPAD)
    acc2 = _edge_pass(adj2d, esc2, ext2, CPAD, CPAD)
    logits_pad = _t3(acc2)
    return (logits_pad[:N, :C], enc_pad[:N])
